# Initial kernel scaffold; baseline (speedup 1.0000x reference)
import jax
import jax.numpy as jnp
from jax.experimental import pallas as pl


def kernel(stem_w2d, stem_b, s0b0_conv1_w2d, s0b0_conv1_b, s0b0_conv2_w2d, s0b0_conv2_b, s0b0_conv3_w2d, s0b0_conv3_b, s0b0_ds_w2d, s0b0_ds_b, s0b1_conv1_w2d, s0b1_conv1_b, s0b1_conv2_w2d, s0b1_conv2_b, s0b1_conv3_w2d, s0b1_conv3_b, s0b2_conv1_w2d, s0b2_conv1_b, s0b2_conv2_w2d, s0b2_conv2_b, s0b2_conv3_w2d, s0b2_conv3_b, s1b0_conv1_w2d, s1b0_conv1_b, s1b0_conv2_w2d, s1b0_conv2_b, s1b0_conv3_w2d, s1b0_conv3_b, s1b0_ds_w2d, s1b0_ds_b, s1b1_conv1_w2d, s1b1_conv1_b, s1b1_conv2_w2d, s1b1_conv2_b, s1b1_conv3_w2d, s1b1_conv3_b, s1b2_conv1_w2d, s1b2_conv1_b, s1b2_conv2_w2d, s1b2_conv2_b, s1b2_conv3_w2d, s1b2_conv3_b, s1b3_conv1_w2d, s1b3_conv1_b, s1b3_conv2_w2d, s1b3_conv2_b, s1b3_conv3_w2d, s1b3_conv3_b, s2b0_conv1_w2d, s2b0_conv1_b, s2b0_conv2_w2d, s2b0_conv2_b, s2b0_conv3_w2d, s2b0_conv3_b, s2b0_ds_w2d, s2b0_ds_b, s2b1_conv1_w2d, s2b1_conv1_b, s2b1_conv2_w2d, s2b1_conv2_b, s2b1_conv3_w2d, s2b1_conv3_b, s2b2_conv1_w2d, s2b2_conv1_b, s2b2_conv2_w2d, s2b2_conv2_b, s2b2_conv3_w2d, s2b2_conv3_b, s2b3_conv1_w2d, s2b3_conv1_b, s2b3_conv2_w2d, s2b3_conv2_b, s2b3_conv3_w2d, s2b3_conv3_b, s2b4_conv1_w2d, s2b4_conv1_b, s2b4_conv2_w2d, s2b4_conv2_b, s2b4_conv3_w2d, s2b4_conv3_b, s2b5_conv1_w2d, s2b5_conv1_b, s2b5_conv2_w2d, s2b5_conv2_b, s2b5_conv3_w2d, s2b5_conv3_b, s3b0_conv1_w2d, s3b0_conv1_b, s3b0_conv2_w2d, s3b0_conv2_b, s3b0_conv3_w2d, s3b0_conv3_b, s3b0_ds_w2d, s3b0_ds_b, s3b1_conv1_w2d, s3b1_conv1_b, s3b1_conv2_w2d, s3b1_conv2_b, s3b1_conv3_w2d, s3b1_conv3_b, s3b2_conv1_w2d, s3b2_conv1_b, s3b2_conv2_w2d, s3b2_conv2_b, s3b2_conv3_w2d, s3b2_conv3_b, fc_w, fc_b, x_nchw):
    raise NotImplementedError("write your pallas kernel here")


# speedup vs baseline: 3.6872x; 3.6872x over previous
import functools

import jax
import jax.numpy as jnp
from jax.experimental import pallas as pl
from jax.experimental.pallas import tpu as pltpu

_CB = 128
_VMEM_LIMIT = 56 * 1024 * 1024


def _block_body(x_ref, w1_ref, b1_ref, w2_ref, b2_ref, w3_ref, b3_ref,
                *rest, G, H, Cin, width, nt, outc, stride, has_ds):
    if has_ds:
        ws_ref, bs_ref = rest[:2]
        rest = rest[2:]
    if stride != 1:
        xs_ref = rest[0]
        rest = rest[1:]
    o_ref, h1p_ref, pat_ref, h2_ref = rest
    Ho = H // stride
    M = G * Ho * Ho
    pdt = jnp.bfloat16 if stride == 1 else jnp.float32

    a = x_ref[...].reshape(G * H * H, Cin)
    h1 = jnp.dot(a, w1_ref[...], preferred_element_type=jnp.float32)
    h1 = jnp.maximum(h1 + b1_ref[...], 0.0)
    if stride == 1:
        h1 = h1.astype(jnp.bfloat16)

    for j in range(nt):
        lo = j * _CB
        hj = h1[:, lo:lo + _CB].astype(pdt).reshape(G, H, H, _CB)
        h1p_ref[:, 0:1, :, :] = jnp.zeros((G, 1, H + 2, _CB), pdt)
        h1p_ref[:, H + 1:H + 2, :, :] = jnp.zeros((G, 1, H + 2, _CB), pdt)
        h1p_ref[:, 1:H + 1, 0:1, :] = jnp.zeros((G, H, 1, _CB), pdt)
        h1p_ref[:, 1:H + 1, H + 1:H + 2, :] = jnp.zeros((G, H, 1, _CB), pdt)
        h1p_ref[:, 1:H + 1, 1:H + 1, :] = hj
        for kh in range(3):
            for kw in range(3):
                t = kh * 3 + kw
                tap = h1p_ref[:, kh:kh + stride * (Ho - 1) + 1:stride,
                              kw:kw + stride * (Ho - 1) + 1:stride, :]
                pat_ref[:, t * _CB:(t + 1) * _CB] = (
                    tap.astype(jnp.bfloat16).reshape(M, _CB))
        acc = jnp.dot(pat_ref[...], w2_ref[j],
                      preferred_element_type=jnp.float32)
        acc = jnp.maximum(acc + b2_ref[:, lo:lo + _CB], 0.0)
        h2_ref[:, lo:lo + _CB] = acc.astype(jnp.bfloat16)

    h3 = jnp.dot(h2_ref[...], w3_ref[...], preferred_element_type=jnp.float32)
    h3 = h3 + b3_ref[...]
    if has_ds:
        if stride != 1:
            xs = xs_ref[...].reshape(M, Cin)
        else:
            xs = x_ref[...].reshape(M, Cin)
        iden = jnp.dot(xs, ws_ref[...], preferred_element_type=jnp.float32)
        iden = (iden + bs_ref[...]).astype(jnp.bfloat16)
        h3 = h3 + iden.astype(jnp.float32)
    else:
        h3 = h3 + x_ref[...].reshape(M, outc).astype(jnp.float32)
    out = jnp.maximum(h3, 0.0).astype(jnp.bfloat16)
    o_ref[...] = out.reshape(G, Ho, Ho, outc)


@functools.lru_cache(maxsize=None)
def _build_block(B, G, H, Cin, width, nt, outc, stride, has_ds):
    Ho = H // stride
    M = G * Ho * Ho
    body = functools.partial(_block_body, G=G, H=H, Cin=Cin, width=width,
                             nt=nt, outc=outc, stride=stride, has_ds=has_ds)
    in_specs = [
        pl.BlockSpec((G, H, H, Cin), lambda i: (i, 0, 0, 0)),
        pl.BlockSpec((Cin, width), lambda i: (0, 0)),
        pl.BlockSpec((1, width), lambda i: (0, 0)),
        pl.BlockSpec((nt, 9 * _CB, _CB), lambda i: (0, 0, 0)),
        pl.BlockSpec((1, width), lambda i: (0, 0)),
        pl.BlockSpec((width, outc), lambda i: (0, 0)),
        pl.BlockSpec((1, outc), lambda i: (0, 0)),
    ]
    if has_ds:
        in_specs.append(pl.BlockSpec((Cin, outc), lambda i: (0, 0)))
        in_specs.append(pl.BlockSpec((1, outc), lambda i: (0, 0)))
    if stride != 1:
        in_specs.append(pl.BlockSpec((G, Ho, Ho, Cin), lambda i: (i, 0, 0, 0)))
    pdt = jnp.bfloat16 if stride == 1 else jnp.float32
    return pl.pallas_call(
        body,
        out_shape=jax.ShapeDtypeStruct((B, Ho, Ho, outc), jnp.bfloat16),
        grid=(B // G,),
        in_specs=in_specs,
        out_specs=pl.BlockSpec((G, Ho, Ho, outc), lambda i: (i, 0, 0, 0)),
        scratch_shapes=[
            pltpu.VMEM((G, H + 2, H + 2, _CB), pdt),
            pltpu.VMEM((M, 9 * _CB), jnp.bfloat16),
            pltpu.VMEM((M, width), jnp.bfloat16),
        ],
        compiler_params=pltpu.CompilerParams(
            dimension_semantics=("parallel",),
            vmem_limit_bytes=_VMEM_LIMIT),
    )


def _run_block(x, w1, b1, w2, b2, w3, b3, ws, bs, *, G, width, nt, outc,
               stride):
    B, H, _, Cin = x.shape
    args = [x, w1, b1.reshape(1, width), w2.reshape(nt, 9 * _CB, _CB),
            b2.reshape(1, width), w3, b3.reshape(1, outc)]
    if ws is not None:
        args += [ws, bs.reshape(1, outc)]
    if stride != 1:
        args.append(x[:, ::stride, ::stride, :])
    call = _build_block(B, G, H, Cin, width, nt, outc, stride, ws is not None)
    return call(*args)


def _stem_body(p_ref, w_ref, b_ref, o_ref, hp_ref):
    p = p_ref[...].reshape(112 * 112, 49)
    h = jnp.dot(p, w_ref[...], preferred_element_type=jnp.float32)
    h = jnp.maximum(h + b_ref[...], 0.0)
    hp_ref[0:1, :, :] = jnp.zeros((1, 114, 64), jnp.float32)
    hp_ref[113:114, :, :] = jnp.zeros((1, 114, 64), jnp.float32)
    hp_ref[1:113, 0:1, :] = jnp.zeros((112, 1, 64), jnp.float32)
    hp_ref[1:113, 113:114, :] = jnp.zeros((112, 1, 64), jnp.float32)
    hp_ref[1:113, 1:113, :] = h.reshape(112, 112, 64)
    m = hp_ref[0:111:2, 0:111:2, :]
    for kh in range(3):
        for kw in range(3):
            if kh == 0 and kw == 0:
                continue
            m = jnp.maximum(m, hp_ref[kh:kh + 111:2, kw:kw + 111:2, :])
    o_ref[...] = m.astype(jnp.bfloat16)[None]


@functools.lru_cache(maxsize=None)
def _build_stem(B):
    return pl.pallas_call(
        _stem_body,
        out_shape=jax.ShapeDtypeStruct((B, 56, 56, 64), jnp.bfloat16),
        grid=(B,),
        in_specs=[
            pl.BlockSpec((1, 112, 112, 49), lambda i: (i, 0, 0, 0)),
            pl.BlockSpec((49, 64), lambda i: (0, 0)),
            pl.BlockSpec((1, 64), lambda i: (0, 0)),
        ],
        out_specs=pl.BlockSpec((1, 56, 56, 64), lambda i: (i, 0, 0, 0)),
        scratch_shapes=[pltpu.VMEM((114, 114, 64), jnp.float32)],
        compiler_params=pltpu.CompilerParams(
            dimension_semantics=("parallel",),
            vmem_limit_bytes=_VMEM_LIMIT),
    )


def _head_body(x_ref, w_ref, b_ref, dense_ref, cls_ref):
    xv = x_ref[...].astype(jnp.float32)
    d = jnp.mean(xv, axis=1)
    dense_ref[...] = d
    cls = jnp.dot(d.astype(jnp.bfloat16), w_ref[...],
                  preferred_element_type=jnp.float32)
    cls_ref[...] = cls + b_ref[...]


@functools.lru_cache(maxsize=None)
def _build_head(B):
    return pl.pallas_call(
        _head_body,
        out_shape=(jax.ShapeDtypeStruct((B, 2048), jnp.float32),
                   jax.ShapeDtypeStruct((B, 6), jnp.float32)),
        grid=(1,),
        in_specs=[
            pl.BlockSpec((B, 49, 2048), lambda i: (0, 0, 0)),
            pl.BlockSpec((2048, 6), lambda i: (0, 0)),
            pl.BlockSpec((1, 6), lambda i: (0, 0)),
        ],
        out_specs=(pl.BlockSpec((B, 2048), lambda i: (0, 0)),
                   pl.BlockSpec((B, 6), lambda i: (0, 0))),
        compiler_params=pltpu.CompilerParams(
            dimension_semantics=("arbitrary",),
            vmem_limit_bytes=_VMEM_LIMIT),
    )


def kernel(stem_w2d, stem_b, s0b0_conv1_w2d, s0b0_conv1_b, s0b0_conv2_w2d, s0b0_conv2_b, s0b0_conv3_w2d, s0b0_conv3_b, s0b0_ds_w2d, s0b0_ds_b, s0b1_conv1_w2d, s0b1_conv1_b, s0b1_conv2_w2d, s0b1_conv2_b, s0b1_conv3_w2d, s0b1_conv3_b, s0b2_conv1_w2d, s0b2_conv1_b, s0b2_conv2_w2d, s0b2_conv2_b, s0b2_conv3_w2d, s0b2_conv3_b, s1b0_conv1_w2d, s1b0_conv1_b, s1b0_conv2_w2d, s1b0_conv2_b, s1b0_conv3_w2d, s1b0_conv3_b, s1b0_ds_w2d, s1b0_ds_b, s1b1_conv1_w2d, s1b1_conv1_b, s1b1_conv2_w2d, s1b1_conv2_b, s1b1_conv3_w2d, s1b1_conv3_b, s1b2_conv1_w2d, s1b2_conv1_b, s1b2_conv2_w2d, s1b2_conv2_b, s1b2_conv3_w2d, s1b2_conv3_b, s1b3_conv1_w2d, s1b3_conv1_b, s1b3_conv2_w2d, s1b3_conv2_b, s1b3_conv3_w2d, s1b3_conv3_b, s2b0_conv1_w2d, s2b0_conv1_b, s2b0_conv2_w2d, s2b0_conv2_b, s2b0_conv3_w2d, s2b0_conv3_b, s2b0_ds_w2d, s2b0_ds_b, s2b1_conv1_w2d, s2b1_conv1_b, s2b1_conv2_w2d, s2b1_conv2_b, s2b1_conv3_w2d, s2b1_conv3_b, s2b2_conv1_w2d, s2b2_conv1_b, s2b2_conv2_w2d, s2b2_conv2_b, s2b2_conv3_w2d, s2b2_conv3_b, s2b3_conv1_w2d, s2b3_conv1_b, s2b3_conv2_w2d, s2b3_conv2_b, s2b3_conv3_w2d, s2b3_conv3_b, s2b4_conv1_w2d, s2b4_conv1_b, s2b4_conv2_w2d, s2b4_conv2_b, s2b4_conv3_w2d, s2b4_conv3_b, s2b5_conv1_w2d, s2b5_conv1_b, s2b5_conv2_w2d, s2b5_conv2_b, s2b5_conv3_w2d, s2b5_conv3_b, s3b0_conv1_w2d, s3b0_conv1_b, s3b0_conv2_w2d, s3b0_conv2_b, s3b0_conv3_w2d, s3b0_conv3_b, s3b0_ds_w2d, s3b0_ds_b, s3b1_conv1_w2d, s3b1_conv1_b, s3b1_conv2_w2d, s3b1_conv2_b, s3b1_conv3_w2d, s3b1_conv3_b, s3b2_conv1_w2d, s3b2_conv1_b, s3b2_conv2_w2d, s3b2_conv2_b, s3b2_conv3_w2d, s3b2_conv3_b, fc_w, fc_b, x_nchw):
    B = x_nchw.shape[0]

    xb = x_nchw.reshape(B, 224, 224).astype(jnp.bfloat16)
    xp = jnp.pad(xb, ((0, 0), (3, 3), (3, 3)))
    cols = []
    for kh in range(7):
        for kw in range(7):
            cols.append(xp[:, kh:kh + 223:2, kw:kw + 223:2])
    patches = jnp.stack(cols, axis=-1)
    x = _build_stem(B)(patches, stem_w2d, stem_b.reshape(1, 64))

    blocks = [
        (s0b0_conv1_w2d, s0b0_conv1_b, s0b0_conv2_w2d, s0b0_conv2_b,
         s0b0_conv3_w2d, s0b0_conv3_b, s0b0_ds_w2d, s0b0_ds_b,
         1, 128, 1, 256, 1),
        (s0b1_conv1_w2d, s0b1_conv1_b, s0b1_conv2_w2d, s0b1_conv2_b,
         s0b1_conv3_w2d, s0b1_conv3_b, None, None, 1, 128, 1, 256, 1),
        (s0b2_conv1_w2d, s0b2_conv1_b, s0b2_conv2_w2d, s0b2_conv2_b,
         s0b2_conv3_w2d, s0b2_conv3_b, None, None, 1, 128, 1, 256, 1),
        (s1b0_conv1_w2d, s1b0_conv1_b, s1b0_conv2_w2d, s1b0_conv2_b,
         s1b0_conv3_w2d, s1b0_conv3_b, s1b0_ds_w2d, s1b0_ds_b,
         2, 256, 2, 512, 2),
        (s1b1_conv1_w2d, s1b1_conv1_b, s1b1_conv2_w2d, s1b1_conv2_b,
         s1b1_conv3_w2d, s1b1_conv3_b, None, None, 2, 256, 2, 512, 1),
        (s1b2_conv1_w2d, s1b2_conv1_b, s1b2_conv2_w2d, s1b2_conv2_b,
         s1b2_conv3_w2d, s1b2_conv3_b, None, None, 2, 256, 2, 512, 1),
        (s1b3_conv1_w2d, s1b3_conv1_b, s1b3_conv2_w2d, s1b3_conv2_b,
         s1b3_conv3_w2d, s1b3_conv3_b, None, None, 2, 256, 2, 512, 1),
        (s2b0_conv1_w2d, s2b0_conv1_b, s2b0_conv2_w2d, s2b0_conv2_b,
         s2b0_conv3_w2d, s2b0_conv3_b, s2b0_ds_w2d, s2b0_ds_b,
         4, 512, 4, 1024, 2),
        (s2b1_conv1_w2d, s2b1_conv1_b, s2b1_conv2_w2d, s2b1_conv2_b,
         s2b1_conv3_w2d, s2b1_conv3_b, None, None, 4, 512, 4, 1024, 1),
        (s2b2_conv1_w2d, s2b2_conv1_b, s2b2_conv2_w2d, s2b2_conv2_b,
         s2b2_conv3_w2d, s2b2_conv3_b, None, None, 4, 512, 4, 1024, 1),
        (s2b3_conv1_w2d, s2b3_conv1_b, s2b3_conv2_w2d, s2b3_conv2_b,
         s2b3_conv3_w2d, s2b3_conv3_b, None, None, 4, 512, 4, 1024, 1),
        (s2b4_conv1_w2d, s2b4_conv1_b, s2b4_conv2_w2d, s2b4_conv2_b,
         s2b4_conv3_w2d, s2b4_conv3_b, None, None, 4, 512, 4, 1024, 1),
        (s2b5_conv1_w2d, s2b5_conv1_b, s2b5_conv2_w2d, s2b5_conv2_b,
         s2b5_conv3_w2d, s2b5_conv3_b, None, None, 4, 512, 4, 1024, 1),
        (s3b0_conv1_w2d, s3b0_conv1_b, s3b0_conv2_w2d, s3b0_conv2_b,
         s3b0_conv3_w2d, s3b0_conv3_b, s3b0_ds_w2d, s3b0_ds_b,
         4, 1024, 8, 2048, 2),
        (s3b1_conv1_w2d, s3b1_conv1_b, s3b1_conv2_w2d, s3b1_conv2_b,
         s3b1_conv3_w2d, s3b1_conv3_b, None, None, 8, 1024, 8, 2048, 1),
        (s3b2_conv1_w2d, s3b2_conv1_b, s3b2_conv2_w2d, s3b2_conv2_b,
         s3b2_conv3_w2d, s3b2_conv3_b, None, None, 8, 1024, 8, 2048, 1),
    ]
    for (w1, b1, w2, b2, w3, b3, ws, bs, G, width, nt, outc, stride) in blocks:
        x = _run_block(x, w1, b1, w2, b2, w3, b3, ws, bs, G=G, width=width,
                       nt=nt, outc=outc, stride=stride)

    dense, cls = _build_head(B)(x.reshape(B, 49, 2048),
                                fc_w.astype(jnp.bfloat16),
                                fc_b.reshape(1, 6))
    return dense, cls


# speedup vs baseline: 11.8492x; 3.2136x over previous
import functools

import jax
import jax.numpy as jnp
from jax.experimental import pallas as pl
from jax.experimental.pallas import tpu as pltpu

_CB = 128
_VMEM_LIMIT = 56 * 1024 * 1024


def _block_body(x_ref, w1_ref, b1_ref, w2_ref, b2_ref, w3_ref, b3_ref,
                *rest, G, H, Cin, width, nt, outc, stride, has_ds):
    if has_ds:
        ws_ref, bs_ref = rest[:2]
        rest = rest[2:]
    if stride != 1:
        xs_ref = rest[0]
        rest = rest[1:]
    o_ref, h1p_ref, pat_ref, h2_ref = rest
    Ho = H // stride
    M = G * Ho * Ho
    pdt = jnp.bfloat16 if stride == 1 else jnp.float32

    a = x_ref[...].reshape(G * H * H, Cin)
    h1 = jnp.dot(a, w1_ref[...], preferred_element_type=jnp.float32)
    h1 = jnp.maximum(h1 + b1_ref[...], 0.0)
    if stride == 1:
        h1 = h1.astype(jnp.bfloat16)

    for j in range(nt):
        lo = j * _CB
        hj = h1[:, lo:lo + _CB].astype(pdt).reshape(G, H, H, _CB)
        h1p_ref[:, 0:1, :, :] = jnp.zeros((G, 1, H + 2, _CB), pdt)
        h1p_ref[:, H + 1:H + 2, :, :] = jnp.zeros((G, 1, H + 2, _CB), pdt)
        h1p_ref[:, 1:H + 1, 0:1, :] = jnp.zeros((G, H, 1, _CB), pdt)
        h1p_ref[:, 1:H + 1, H + 1:H + 2, :] = jnp.zeros((G, H, 1, _CB), pdt)
        h1p_ref[:, 1:H + 1, 1:H + 1, :] = hj
        for kh in range(3):
            for kw in range(3):
                t = kh * 3 + kw
                tap = h1p_ref[:, kh:kh + stride * (Ho - 1) + 1:stride,
                              kw:kw + stride * (Ho - 1) + 1:stride, :]
                pat_ref[:, t * _CB:(t + 1) * _CB] = (
                    tap.astype(jnp.bfloat16).reshape(M, _CB))
        acc = jnp.dot(pat_ref[...], w2_ref[j],
                      preferred_element_type=jnp.float32)
        acc = jnp.maximum(acc + b2_ref[:, lo:lo + _CB], 0.0)
        h2_ref[:, lo:lo + _CB] = acc.astype(jnp.bfloat16)

    h3 = jnp.dot(h2_ref[...], w3_ref[...], preferred_element_type=jnp.float32)
    h3 = h3 + b3_ref[...]
    if has_ds:
        if stride != 1:
            xs = xs_ref[...].reshape(M, Cin)
        else:
            xs = x_ref[...].reshape(M, Cin)
        iden = jnp.dot(xs, ws_ref[...], preferred_element_type=jnp.float32)
        iden = (iden + bs_ref[...]).astype(jnp.bfloat16)
        h3 = h3 + iden.astype(jnp.float32)
    else:
        h3 = h3 + x_ref[...].reshape(M, outc).astype(jnp.float32)
    out = jnp.maximum(h3, 0.0).astype(jnp.bfloat16)
    o_ref[...] = out.reshape(G, Ho, Ho, outc)


@functools.lru_cache(maxsize=None)
def _build_block(B, G, H, Cin, width, nt, outc, stride, has_ds):
    Ho = H // stride
    M = G * Ho * Ho
    body = functools.partial(_block_body, G=G, H=H, Cin=Cin, width=width,
                             nt=nt, outc=outc, stride=stride, has_ds=has_ds)
    in_specs = [
        pl.BlockSpec((G, H, H, Cin), lambda i: (i, 0, 0, 0)),
        pl.BlockSpec((Cin, width), lambda i: (0, 0)),
        pl.BlockSpec((1, width), lambda i: (0, 0)),
        pl.BlockSpec((nt, 9 * _CB, _CB), lambda i: (0, 0, 0)),
        pl.BlockSpec((1, width), lambda i: (0, 0)),
        pl.BlockSpec((width, outc), lambda i: (0, 0)),
        pl.BlockSpec((1, outc), lambda i: (0, 0)),
    ]
    if has_ds:
        in_specs.append(pl.BlockSpec((Cin, outc), lambda i: (0, 0)))
        in_specs.append(pl.BlockSpec((1, outc), lambda i: (0, 0)))
    if stride != 1:
        in_specs.append(pl.BlockSpec((G, Ho, Ho, Cin), lambda i: (i, 0, 0, 0)))
    pdt = jnp.bfloat16 if stride == 1 else jnp.float32
    return pl.pallas_call(
        body,
        out_shape=jax.ShapeDtypeStruct((B, Ho, Ho, outc), jnp.bfloat16),
        grid=(B // G,),
        in_specs=in_specs,
        out_specs=pl.BlockSpec((G, Ho, Ho, outc), lambda i: (i, 0, 0, 0)),
        scratch_shapes=[
            pltpu.VMEM((G, H + 2, H + 2, _CB), pdt),
            pltpu.VMEM((M, 9 * _CB), jnp.bfloat16),
            pltpu.VMEM((M, width), jnp.bfloat16),
        ],
        compiler_params=pltpu.CompilerParams(
            dimension_semantics=("parallel",),
            vmem_limit_bytes=_VMEM_LIMIT),
    )


def _run_block(x, w1, b1, w2, b2, w3, b3, ws, bs, *, G, width, nt, outc,
               stride):
    B, H, _, Cin = x.shape
    args = [x, w1, b1.reshape(1, width), w2.reshape(nt, 9 * _CB, _CB),
            b2.reshape(1, width), w3, b3.reshape(1, outc)]
    if ws is not None:
        args += [ws, bs.reshape(1, outc)]
    if stride != 1:
        args.append(x[:, ::stride, ::stride, :])
    call = _build_block(B, G, H, Cin, width, nt, outc, stride, ws is not None)
    return call(*args)


def _stem_body(p_ref, w_ref, b_ref, o_ref, hp_ref):
    p = p_ref[0]
    h = jax.lax.dot_general(p, w_ref[...], (((0,), (0,)), ((), ())),
                            preferred_element_type=jnp.float32)
    h = jnp.maximum(h + b_ref[...], 0.0)
    hp_ref[0:1, :, :] = jnp.zeros((1, 114, 64), jnp.float32)
    hp_ref[113:114, :, :] = jnp.zeros((1, 114, 64), jnp.float32)
    hp_ref[1:113, 0:1, :] = jnp.zeros((112, 1, 64), jnp.float32)
    hp_ref[1:113, 113:114, :] = jnp.zeros((112, 1, 64), jnp.float32)
    hp_ref[1:113, 1:113, :] = h.reshape(112, 112, 64)
    m = hp_ref[0:111:2, 0:111:2, :]
    for kh in range(3):
        for kw in range(3):
            if kh == 0 and kw == 0:
                continue
            m = jnp.maximum(m, hp_ref[kh:kh + 111:2, kw:kw + 111:2, :])
    o_ref[...] = m.astype(jnp.bfloat16)[None]


@functools.lru_cache(maxsize=None)
def _build_stem(B):
    return pl.pallas_call(
        _stem_body,
        out_shape=jax.ShapeDtypeStruct((B, 56, 56, 64), jnp.bfloat16),
        grid=(B,),
        in_specs=[
            pl.BlockSpec((1, 49, 112 * 112), lambda i: (i, 0, 0)),
            pl.BlockSpec((49, 64), lambda i: (0, 0)),
            pl.BlockSpec((1, 64), lambda i: (0, 0)),
        ],
        out_specs=pl.BlockSpec((1, 56, 56, 64), lambda i: (i, 0, 0, 0)),
        scratch_shapes=[pltpu.VMEM((114, 114, 64), jnp.float32)],
        compiler_params=pltpu.CompilerParams(
            dimension_semantics=("parallel",),
            vmem_limit_bytes=_VMEM_LIMIT),
    )


def _head_body(x_ref, w_ref, b_ref, dense_ref, cls_ref):
    xv = x_ref[...].astype(jnp.float32)
    d = jnp.mean(xv, axis=1)
    dense_ref[...] = d
    cls = jnp.dot(d.astype(jnp.bfloat16), w_ref[...],
                  preferred_element_type=jnp.float32)
    cls_ref[...] = cls + b_ref[...]


@functools.lru_cache(maxsize=None)
def _build_head(B):
    return pl.pallas_call(
        _head_body,
        out_shape=(jax.ShapeDtypeStruct((B, 2048), jnp.float32),
                   jax.ShapeDtypeStruct((B, 6), jnp.float32)),
        grid=(1,),
        in_specs=[
            pl.BlockSpec((B, 49, 2048), lambda i: (0, 0, 0)),
            pl.BlockSpec((2048, 6), lambda i: (0, 0)),
            pl.BlockSpec((1, 6), lambda i: (0, 0)),
        ],
        out_specs=(pl.BlockSpec((B, 2048), lambda i: (0, 0)),
                   pl.BlockSpec((B, 6), lambda i: (0, 0))),
        compiler_params=pltpu.CompilerParams(
            dimension_semantics=("arbitrary",),
            vmem_limit_bytes=_VMEM_LIMIT),
    )


def kernel(stem_w2d, stem_b, s0b0_conv1_w2d, s0b0_conv1_b, s0b0_conv2_w2d, s0b0_conv2_b, s0b0_conv3_w2d, s0b0_conv3_b, s0b0_ds_w2d, s0b0_ds_b, s0b1_conv1_w2d, s0b1_conv1_b, s0b1_conv2_w2d, s0b1_conv2_b, s0b1_conv3_w2d, s0b1_conv3_b, s0b2_conv1_w2d, s0b2_conv1_b, s0b2_conv2_w2d, s0b2_conv2_b, s0b2_conv3_w2d, s0b2_conv3_b, s1b0_conv1_w2d, s1b0_conv1_b, s1b0_conv2_w2d, s1b0_conv2_b, s1b0_conv3_w2d, s1b0_conv3_b, s1b0_ds_w2d, s1b0_ds_b, s1b1_conv1_w2d, s1b1_conv1_b, s1b1_conv2_w2d, s1b1_conv2_b, s1b1_conv3_w2d, s1b1_conv3_b, s1b2_conv1_w2d, s1b2_conv1_b, s1b2_conv2_w2d, s1b2_conv2_b, s1b2_conv3_w2d, s1b2_conv3_b, s1b3_conv1_w2d, s1b3_conv1_b, s1b3_conv2_w2d, s1b3_conv2_b, s1b3_conv3_w2d, s1b3_conv3_b, s2b0_conv1_w2d, s2b0_conv1_b, s2b0_conv2_w2d, s2b0_conv2_b, s2b0_conv3_w2d, s2b0_conv3_b, s2b0_ds_w2d, s2b0_ds_b, s2b1_conv1_w2d, s2b1_conv1_b, s2b1_conv2_w2d, s2b1_conv2_b, s2b1_conv3_w2d, s2b1_conv3_b, s2b2_conv1_w2d, s2b2_conv1_b, s2b2_conv2_w2d, s2b2_conv2_b, s2b2_conv3_w2d, s2b2_conv3_b, s2b3_conv1_w2d, s2b3_conv1_b, s2b3_conv2_w2d, s2b3_conv2_b, s2b3_conv3_w2d, s2b3_conv3_b, s2b4_conv1_w2d, s2b4_conv1_b, s2b4_conv2_w2d, s2b4_conv2_b, s2b4_conv3_w2d, s2b4_conv3_b, s2b5_conv1_w2d, s2b5_conv1_b, s2b5_conv2_w2d, s2b5_conv2_b, s2b5_conv3_w2d, s2b5_conv3_b, s3b0_conv1_w2d, s3b0_conv1_b, s3b0_conv2_w2d, s3b0_conv2_b, s3b0_conv3_w2d, s3b0_conv3_b, s3b0_ds_w2d, s3b0_ds_b, s3b1_conv1_w2d, s3b1_conv1_b, s3b1_conv2_w2d, s3b1_conv2_b, s3b1_conv3_w2d, s3b1_conv3_b, s3b2_conv1_w2d, s3b2_conv1_b, s3b2_conv2_w2d, s3b2_conv2_b, s3b2_conv3_w2d, s3b2_conv3_b, fc_w, fc_b, x_nchw):
    B = x_nchw.shape[0]

    xb = x_nchw.reshape(B, 224, 224).astype(jnp.bfloat16)
    xpad = jnp.pad(xb, ((0, 0), (3, 3), (3, 3)))
    par = [[xpad[:, p::2, q::2] for q in (0, 1)] for p in (0, 1)]
    cols = []
    for kh in range(7):
        for kw in range(7):
            cols.append(par[kh % 2][kw % 2][:, kh // 2:kh // 2 + 112,
                                            kw // 2:kw // 2 + 112])
    patches = jnp.stack(cols, axis=1).reshape(B, 49, 112 * 112)
    x = _build_stem(B)(patches, stem_w2d, stem_b.reshape(1, 64))

    blocks = [
        (s0b0_conv1_w2d, s0b0_conv1_b, s0b0_conv2_w2d, s0b0_conv2_b,
         s0b0_conv3_w2d, s0b0_conv3_b, s0b0_ds_w2d, s0b0_ds_b,
         1, 128, 1, 256, 1),
        (s0b1_conv1_w2d, s0b1_conv1_b, s0b1_conv2_w2d, s0b1_conv2_b,
         s0b1_conv3_w2d, s0b1_conv3_b, None, None, 1, 128, 1, 256, 1),
        (s0b2_conv1_w2d, s0b2_conv1_b, s0b2_conv2_w2d, s0b2_conv2_b,
         s0b2_conv3_w2d, s0b2_conv3_b, None, None, 1, 128, 1, 256, 1),
        (s1b0_conv1_w2d, s1b0_conv1_b, s1b0_conv2_w2d, s1b0_conv2_b,
         s1b0_conv3_w2d, s1b0_conv3_b, s1b0_ds_w2d, s1b0_ds_b,
         2, 256, 2, 512, 2),
        (s1b1_conv1_w2d, s1b1_conv1_b, s1b1_conv2_w2d, s1b1_conv2_b,
         s1b1_conv3_w2d, s1b1_conv3_b, None, None, 2, 256, 2, 512, 1),
        (s1b2_conv1_w2d, s1b2_conv1_b, s1b2_conv2_w2d, s1b2_conv2_b,
         s1b2_conv3_w2d, s1b2_conv3_b, None, None, 2, 256, 2, 512, 1),
        (s1b3_conv1_w2d, s1b3_conv1_b, s1b3_conv2_w2d, s1b3_conv2_b,
         s1b3_conv3_w2d, s1b3_conv3_b, None, None, 2, 256, 2, 512, 1),
        (s2b0_conv1_w2d, s2b0_conv1_b, s2b0_conv2_w2d, s2b0_conv2_b,
         s2b0_conv3_w2d, s2b0_conv3_b, s2b0_ds_w2d, s2b0_ds_b,
         4, 512, 4, 1024, 2),
        (s2b1_conv1_w2d, s2b1_conv1_b, s2b1_conv2_w2d, s2b1_conv2_b,
         s2b1_conv3_w2d, s2b1_conv3_b, None, None, 4, 512, 4, 1024, 1),
        (s2b2_conv1_w2d, s2b2_conv1_b, s2b2_conv2_w2d, s2b2_conv2_b,
         s2b2_conv3_w2d, s2b2_conv3_b, None, None, 4, 512, 4, 1024, 1),
        (s2b3_conv1_w2d, s2b3_conv1_b, s2b3_conv2_w2d, s2b3_conv2_b,
         s2b3_conv3_w2d, s2b3_conv3_b, None, None, 4, 512, 4, 1024, 1),
        (s2b4_conv1_w2d, s2b4_conv1_b, s2b4_conv2_w2d, s2b4_conv2_b,
         s2b4_conv3_w2d, s2b4_conv3_b, None, None, 4, 512, 4, 1024, 1),
        (s2b5_conv1_w2d, s2b5_conv1_b, s2b5_conv2_w2d, s2b5_conv2_b,
         s2b5_conv3_w2d, s2b5_conv3_b, None, None, 4, 512, 4, 1024, 1),
        (s3b0_conv1_w2d, s3b0_conv1_b, s3b0_conv2_w2d, s3b0_conv2_b,
         s3b0_conv3_w2d, s3b0_conv3_b, s3b0_ds_w2d, s3b0_ds_b,
         4, 1024, 8, 2048, 2),
        (s3b1_conv1_w2d, s3b1_conv1_b, s3b1_conv2_w2d, s3b1_conv2_b,
         s3b1_conv3_w2d, s3b1_conv3_b, None, None, 8, 1024, 8, 2048, 1),
        (s3b2_conv1_w2d, s3b2_conv1_b, s3b2_conv2_w2d, s3b2_conv2_b,
         s3b2_conv3_w2d, s3b2_conv3_b, None, None, 8, 1024, 8, 2048, 1),
    ]
    for (w1, b1, w2, b2, w3, b3, ws, bs, G, width, nt, outc, stride) in blocks:
        x = _run_block(x, w1, b1, w2, b2, w3, b3, ws, bs, G=G, width=width,
                       nt=nt, outc=outc, stride=stride)

    dense, cls = _build_head(B)(x.reshape(B, 49, 2048),
                                fc_w.astype(jnp.bfloat16),
                                fc_b.reshape(1, 6))
    return dense, cls


# speedup vs baseline: 12.9176x; 1.0902x over previous
import functools

import jax
import jax.numpy as jnp
from jax.experimental import pallas as pl
from jax.experimental.pallas import tpu as pltpu

_CB = 128
_VMEM_LIMIT = 56 * 1024 * 1024


def _block_body(x_ref, w1_ref, b1_ref, w2_ref, b2_ref, w3_ref, b3_ref,
                *rest, G, H, Cin, width, nt, outc, stride, has_ds):
    if has_ds:
        ws_ref, bs_ref = rest[:2]
        rest = rest[2:]
    if stride != 1:
        xs_ref = rest[0]
        rest = rest[1:]
    o_ref, h1p_ref, s_ref, h2_ref = rest
    Ho = H // stride
    M = G * Ho * Ho
    pdt = jnp.bfloat16 if stride == 1 else jnp.float32

    a = x_ref[...].reshape(G * H * H, Cin)
    h1 = jnp.dot(a, w1_ref[...], preferred_element_type=jnp.float32)
    h1 = jnp.maximum(h1 + b1_ref[...], 0.0)
    if stride == 1:
        h1 = h1.astype(jnp.bfloat16)

    for j in range(nt):
        lo = j * _CB
        hj = h1[:, lo:lo + _CB].astype(pdt).reshape(G, H, H, _CB)
        h1p_ref[:, 0:1, :, :] = jnp.zeros((G, 1, H + 2, _CB), pdt)
        h1p_ref[:, H + 1:H + 2, :, :] = jnp.zeros((G, 1, H + 2, _CB), pdt)
        h1p_ref[:, 1:H + 1, 0:1, :] = jnp.zeros((G, H, 1, _CB), pdt)
        h1p_ref[:, 1:H + 1, H + 1:H + 2, :] = jnp.zeros((G, H, 1, _CB), pdt)
        h1p_ref[:, 1:H + 1, 1:H + 1, :] = hj
        for kw in range(3):
            s_ref[kw] = h1p_ref[:, :, kw:kw + stride * (Ho - 1) + 1:stride, :]
        acc = None
        for kh in range(3):
            for kw in range(3):
                t = kh * 3 + kw
                tap = s_ref[kw, :, kh:kh + stride * (Ho - 1) + 1:stride, :, :]
                tap = tap.astype(jnp.bfloat16).reshape(M, _CB)
                d = jnp.dot(tap, w2_ref[j, t * _CB:(t + 1) * _CB, :],
                            preferred_element_type=jnp.float32)
                acc = d if acc is None else acc + d
        acc = jnp.maximum(acc + b2_ref[:, lo:lo + _CB], 0.0)
        h2_ref[:, lo:lo + _CB] = acc.astype(jnp.bfloat16)

    h3 = jnp.dot(h2_ref[...], w3_ref[...], preferred_element_type=jnp.float32)
    h3 = h3 + b3_ref[...]
    if has_ds:
        if stride != 1:
            xs = xs_ref[...].reshape(M, Cin)
        else:
            xs = x_ref[...].reshape(M, Cin)
        iden = jnp.dot(xs, ws_ref[...], preferred_element_type=jnp.float32)
        iden = (iden + bs_ref[...]).astype(jnp.bfloat16)
        h3 = h3 + iden.astype(jnp.float32)
    else:
        h3 = h3 + x_ref[...].reshape(M, outc).astype(jnp.float32)
    out = jnp.maximum(h3, 0.0).astype(jnp.bfloat16)
    o_ref[...] = out.reshape(G, Ho, Ho, outc)


@functools.lru_cache(maxsize=None)
def _build_block(B, G, H, Cin, width, nt, outc, stride, has_ds):
    Ho = H // stride
    M = G * Ho * Ho
    body = functools.partial(_block_body, G=G, H=H, Cin=Cin, width=width,
                             nt=nt, outc=outc, stride=stride, has_ds=has_ds)
    in_specs = [
        pl.BlockSpec((G, H, H, Cin), lambda i: (i, 0, 0, 0)),
        pl.BlockSpec((Cin, width), lambda i: (0, 0)),
        pl.BlockSpec((1, width), lambda i: (0, 0)),
        pl.BlockSpec((nt, 9 * _CB, _CB), lambda i: (0, 0, 0)),
        pl.BlockSpec((1, width), lambda i: (0, 0)),
        pl.BlockSpec((width, outc), lambda i: (0, 0)),
        pl.BlockSpec((1, outc), lambda i: (0, 0)),
    ]
    if has_ds:
        in_specs.append(pl.BlockSpec((Cin, outc), lambda i: (0, 0)))
        in_specs.append(pl.BlockSpec((1, outc), lambda i: (0, 0)))
    if stride != 1:
        in_specs.append(pl.BlockSpec((G, Ho, Ho, Cin), lambda i: (i, 0, 0, 0)))
    pdt = jnp.bfloat16 if stride == 1 else jnp.float32
    return pl.pallas_call(
        body,
        out_shape=jax.ShapeDtypeStruct((B, Ho, Ho, outc), jnp.bfloat16),
        grid=(B // G,),
        in_specs=in_specs,
        out_specs=pl.BlockSpec((G, Ho, Ho, outc), lambda i: (i, 0, 0, 0)),
        scratch_shapes=[
            pltpu.VMEM((G, H + 2, H + 2, _CB), pdt),
            pltpu.VMEM((3, G, H + 2, Ho, _CB), pdt),
            pltpu.VMEM((M, width), jnp.bfloat16),
        ],
        compiler_params=pltpu.CompilerParams(
            dimension_semantics=("parallel",),
            vmem_limit_bytes=_VMEM_LIMIT),
    )


def _run_block(x, w1, b1, w2, b2, w3, b3, ws, bs, *, G, width, nt, outc,
               stride):
    B, H, _, Cin = x.shape
    args = [x, w1, b1.reshape(1, width), w2.reshape(nt, 9 * _CB, _CB),
            b2.reshape(1, width), w3, b3.reshape(1, outc)]
    if ws is not None:
        args += [ws, bs.reshape(1, outc)]
    if stride != 1:
        args.append(x[:, ::stride, ::stride, :])
    call = _build_block(B, G, H, Cin, width, nt, outc, stride, ws is not None)
    return call(*args)


def _stem_body(p_ref, w_ref, b_ref, o_ref, hp_ref):
    p = p_ref[0]
    h = jax.lax.dot_general(p, w_ref[...], (((0,), (0,)), ((), ())),
                            preferred_element_type=jnp.float32)
    h = jnp.maximum(h + b_ref[...], 0.0)
    hp_ref[0:1, :, :] = jnp.zeros((1, 114, 64), jnp.float32)
    hp_ref[113:114, :, :] = jnp.zeros((1, 114, 64), jnp.float32)
    hp_ref[1:113, 0:1, :] = jnp.zeros((112, 1, 64), jnp.float32)
    hp_ref[1:113, 113:114, :] = jnp.zeros((112, 1, 64), jnp.float32)
    hp_ref[1:113, 1:113, :] = h.reshape(112, 112, 64)
    m = hp_ref[0:111:2, 0:111:2, :]
    for kh in range(3):
        for kw in range(3):
            if kh == 0 and kw == 0:
                continue
            m = jnp.maximum(m, hp_ref[kh:kh + 111:2, kw:kw + 111:2, :])
    o_ref[...] = m.astype(jnp.bfloat16)[None]


@functools.lru_cache(maxsize=None)
def _build_stem(B):
    return pl.pallas_call(
        _stem_body,
        out_shape=jax.ShapeDtypeStruct((B, 56, 56, 64), jnp.bfloat16),
        grid=(B,),
        in_specs=[
            pl.BlockSpec((1, 49, 112 * 112), lambda i: (i, 0, 0)),
            pl.BlockSpec((49, 64), lambda i: (0, 0)),
            pl.BlockSpec((1, 64), lambda i: (0, 0)),
        ],
        out_specs=pl.BlockSpec((1, 56, 56, 64), lambda i: (i, 0, 0, 0)),
        scratch_shapes=[pltpu.VMEM((114, 114, 64), jnp.float32)],
        compiler_params=pltpu.CompilerParams(
            dimension_semantics=("parallel",),
            vmem_limit_bytes=_VMEM_LIMIT),
    )


def _head_body(x_ref, w_ref, b_ref, dense_ref, cls_ref):
    xv = x_ref[...].astype(jnp.float32)
    d = jnp.mean(xv, axis=1)
    dense_ref[...] = d
    cls = jnp.dot(d.astype(jnp.bfloat16), w_ref[...],
                  preferred_element_type=jnp.float32)
    cls_ref[...] = cls + b_ref[...]


@functools.lru_cache(maxsize=None)
def _build_head(B):
    return pl.pallas_call(
        _head_body,
        out_shape=(jax.ShapeDtypeStruct((B, 2048), jnp.float32),
                   jax.ShapeDtypeStruct((B, 6), jnp.float32)),
        grid=(1,),
        in_specs=[
            pl.BlockSpec((B, 49, 2048), lambda i: (0, 0, 0)),
            pl.BlockSpec((2048, 6), lambda i: (0, 0)),
            pl.BlockSpec((1, 6), lambda i: (0, 0)),
        ],
        out_specs=(pl.BlockSpec((B, 2048), lambda i: (0, 0)),
                   pl.BlockSpec((B, 6), lambda i: (0, 0))),
        compiler_params=pltpu.CompilerParams(
            dimension_semantics=("arbitrary",),
            vmem_limit_bytes=_VMEM_LIMIT),
    )


def kernel(stem_w2d, stem_b, s0b0_conv1_w2d, s0b0_conv1_b, s0b0_conv2_w2d, s0b0_conv2_b, s0b0_conv3_w2d, s0b0_conv3_b, s0b0_ds_w2d, s0b0_ds_b, s0b1_conv1_w2d, s0b1_conv1_b, s0b1_conv2_w2d, s0b1_conv2_b, s0b1_conv3_w2d, s0b1_conv3_b, s0b2_conv1_w2d, s0b2_conv1_b, s0b2_conv2_w2d, s0b2_conv2_b, s0b2_conv3_w2d, s0b2_conv3_b, s1b0_conv1_w2d, s1b0_conv1_b, s1b0_conv2_w2d, s1b0_conv2_b, s1b0_conv3_w2d, s1b0_conv3_b, s1b0_ds_w2d, s1b0_ds_b, s1b1_conv1_w2d, s1b1_conv1_b, s1b1_conv2_w2d, s1b1_conv2_b, s1b1_conv3_w2d, s1b1_conv3_b, s1b2_conv1_w2d, s1b2_conv1_b, s1b2_conv2_w2d, s1b2_conv2_b, s1b2_conv3_w2d, s1b2_conv3_b, s1b3_conv1_w2d, s1b3_conv1_b, s1b3_conv2_w2d, s1b3_conv2_b, s1b3_conv3_w2d, s1b3_conv3_b, s2b0_conv1_w2d, s2b0_conv1_b, s2b0_conv2_w2d, s2b0_conv2_b, s2b0_conv3_w2d, s2b0_conv3_b, s2b0_ds_w2d, s2b0_ds_b, s2b1_conv1_w2d, s2b1_conv1_b, s2b1_conv2_w2d, s2b1_conv2_b, s2b1_conv3_w2d, s2b1_conv3_b, s2b2_conv1_w2d, s2b2_conv1_b, s2b2_conv2_w2d, s2b2_conv2_b, s2b2_conv3_w2d, s2b2_conv3_b, s2b3_conv1_w2d, s2b3_conv1_b, s2b3_conv2_w2d, s2b3_conv2_b, s2b3_conv3_w2d, s2b3_conv3_b, s2b4_conv1_w2d, s2b4_conv1_b, s2b4_conv2_w2d, s2b4_conv2_b, s2b4_conv3_w2d, s2b4_conv3_b, s2b5_conv1_w2d, s2b5_conv1_b, s2b5_conv2_w2d, s2b5_conv2_b, s2b5_conv3_w2d, s2b5_conv3_b, s3b0_conv1_w2d, s3b0_conv1_b, s3b0_conv2_w2d, s3b0_conv2_b, s3b0_conv3_w2d, s3b0_conv3_b, s3b0_ds_w2d, s3b0_ds_b, s3b1_conv1_w2d, s3b1_conv1_b, s3b1_conv2_w2d, s3b1_conv2_b, s3b1_conv3_w2d, s3b1_conv3_b, s3b2_conv1_w2d, s3b2_conv1_b, s3b2_conv2_w2d, s3b2_conv2_b, s3b2_conv3_w2d, s3b2_conv3_b, fc_w, fc_b, x_nchw):
    B = x_nchw.shape[0]

    xb = x_nchw.reshape(B, 224, 224).astype(jnp.bfloat16)
    xpad = jnp.pad(xb, ((0, 0), (3, 3), (3, 3)))
    par = [[xpad[:, p::2, q::2] for q in (0, 1)] for p in (0, 1)]
    cols = []
    for kh in range(7):
        for kw in range(7):
            cols.append(par[kh % 2][kw % 2][:, kh // 2:kh // 2 + 112,
                                            kw // 2:kw // 2 + 112])
    patches = jnp.stack(cols, axis=1).reshape(B, 49, 112 * 112)
    x = _build_stem(B)(patches, stem_w2d, stem_b.reshape(1, 64))

    blocks = [
        (s0b0_conv1_w2d, s0b0_conv1_b, s0b0_conv2_w2d, s0b0_conv2_b,
         s0b0_conv3_w2d, s0b0_conv3_b, s0b0_ds_w2d, s0b0_ds_b,
         1, 128, 1, 256, 1),
        (s0b1_conv1_w2d, s0b1_conv1_b, s0b1_conv2_w2d, s0b1_conv2_b,
         s0b1_conv3_w2d, s0b1_conv3_b, None, None, 1, 128, 1, 256, 1),
        (s0b2_conv1_w2d, s0b2_conv1_b, s0b2_conv2_w2d, s0b2_conv2_b,
         s0b2_conv3_w2d, s0b2_conv3_b, None, None, 1, 128, 1, 256, 1),
        (s1b0_conv1_w2d, s1b0_conv1_b, s1b0_conv2_w2d, s1b0_conv2_b,
         s1b0_conv3_w2d, s1b0_conv3_b, s1b0_ds_w2d, s1b0_ds_b,
         2, 256, 2, 512, 2),
        (s1b1_conv1_w2d, s1b1_conv1_b, s1b1_conv2_w2d, s1b1_conv2_b,
         s1b1_conv3_w2d, s1b1_conv3_b, None, None, 2, 256, 2, 512, 1),
        (s1b2_conv1_w2d, s1b2_conv1_b, s1b2_conv2_w2d, s1b2_conv2_b,
         s1b2_conv3_w2d, s1b2_conv3_b, None, None, 2, 256, 2, 512, 1),
        (s1b3_conv1_w2d, s1b3_conv1_b, s1b3_conv2_w2d, s1b3_conv2_b,
         s1b3_conv3_w2d, s1b3_conv3_b, None, None, 2, 256, 2, 512, 1),
        (s2b0_conv1_w2d, s2b0_conv1_b, s2b0_conv2_w2d, s2b0_conv2_b,
         s2b0_conv3_w2d, s2b0_conv3_b, s2b0_ds_w2d, s2b0_ds_b,
         4, 512, 4, 1024, 2),
        (s2b1_conv1_w2d, s2b1_conv1_b, s2b1_conv2_w2d, s2b1_conv2_b,
         s2b1_conv3_w2d, s2b1_conv3_b, None, None, 4, 512, 4, 1024, 1),
        (s2b2_conv1_w2d, s2b2_conv1_b, s2b2_conv2_w2d, s2b2_conv2_b,
         s2b2_conv3_w2d, s2b2_conv3_b, None, None, 4, 512, 4, 1024, 1),
        (s2b3_conv1_w2d, s2b3_conv1_b, s2b3_conv2_w2d, s2b3_conv2_b,
         s2b3_conv3_w2d, s2b3_conv3_b, None, None, 4, 512, 4, 1024, 1),
        (s2b4_conv1_w2d, s2b4_conv1_b, s2b4_conv2_w2d, s2b4_conv2_b,
         s2b4_conv3_w2d, s2b4_conv3_b, None, None, 4, 512, 4, 1024, 1),
        (s2b5_conv1_w2d, s2b5_conv1_b, s2b5_conv2_w2d, s2b5_conv2_b,
         s2b5_conv3_w2d, s2b5_conv3_b, None, None, 4, 512, 4, 1024, 1),
        (s3b0_conv1_w2d, s3b0_conv1_b, s3b0_conv2_w2d, s3b0_conv2_b,
         s3b0_conv3_w2d, s3b0_conv3_b, s3b0_ds_w2d, s3b0_ds_b,
         4, 1024, 8, 2048, 2),
        (s3b1_conv1_w2d, s3b1_conv1_b, s3b1_conv2_w2d, s3b1_conv2_b,
         s3b1_conv3_w2d, s3b1_conv3_b, None, None, 8, 1024, 8, 2048, 1),
        (s3b2_conv1_w2d, s3b2_conv1_b, s3b2_conv2_w2d, s3b2_conv2_b,
         s3b2_conv3_w2d, s3b2_conv3_b, None, None, 8, 1024, 8, 2048, 1),
    ]
    for (w1, b1, w2, b2, w3, b3, ws, bs, G, width, nt, outc, stride) in blocks:
        x = _run_block(x, w1, b1, w2, b2, w3, b3, ws, bs, G=G, width=width,
                       nt=nt, outc=outc, stride=stride)

    dense, cls = _build_head(B)(x.reshape(B, 49, 2048),
                                fc_w.astype(jnp.bfloat16),
                                fc_b.reshape(1, 6))
    return dense, cls


# speedup vs baseline: 12.9352x; 1.0014x over previous
import functools

import jax
import jax.numpy as jnp
from jax.experimental import pallas as pl
from jax.experimental.pallas import tpu as pltpu

_CB = 128
_VMEM_LIMIT = 56 * 1024 * 1024


def _bneck(xc, xs, wr, sc, G, H, Cin, width, nt, outc, stride):
    Ho = H // stride
    M = G * Ho * Ho
    pdt = jnp.bfloat16 if stride == 1 else jnp.float32
    h1p_ref = sc['h1pb'] if stride == 1 else sc['h1p32']
    s_ref = sc['sb'] if stride == 1 else sc['s32']
    h2_ref = sc['h2']

    w1, b1, w2, b2, w3, b3 = wr[:6]
    h1 = jnp.dot(xc, w1[...], preferred_element_type=jnp.float32)
    h1 = jnp.maximum(h1 + b1[...], 0.0)
    if stride == 1:
        h1 = h1.astype(jnp.bfloat16)

    for j in range(nt):
        lo = j * _CB
        hj = h1[:, lo:lo + _CB].astype(pdt).reshape(G, H, H, _CB)
        h1p_ref[:, 0:1, :, :] = jnp.zeros((G, 1, H + 2, _CB), pdt)
        h1p_ref[:, H + 1:H + 2, :, :] = jnp.zeros((G, 1, H + 2, _CB), pdt)
        h1p_ref[:, 1:H + 1, 0:1, :] = jnp.zeros((G, H, 1, _CB), pdt)
        h1p_ref[:, 1:H + 1, H + 1:H + 2, :] = jnp.zeros((G, H, 1, _CB), pdt)
        h1p_ref[:, 1:H + 1, 1:H + 1, :] = hj
        for kw in range(3):
            s_ref[kw] = h1p_ref[:, :, kw:kw + stride * (Ho - 1) + 1:stride, :]
        acc = None
        for kh in range(3):
            for kw in range(3):
                t = kh * 3 + kw
                tap = s_ref[kw, :, kh:kh + stride * (Ho - 1) + 1:stride, :, :]
                tap = tap.astype(jnp.bfloat16).reshape(M, _CB)
                d = jnp.dot(tap, w2[j, t * _CB:(t + 1) * _CB, :],
                            preferred_element_type=jnp.float32)
                acc = d if acc is None else acc + d
        acc = jnp.maximum(acc + b2[:, lo:lo + _CB], 0.0)
        h2_ref[:, lo:lo + _CB] = acc.astype(jnp.bfloat16)

    h3 = jnp.dot(h2_ref[:, :width], w3[...],
                 preferred_element_type=jnp.float32)
    h3 = h3 + b3[...]
    if len(wr) > 6:
        ws, bs = wr[6:8]
        iden = jnp.dot(xs if xs is not None else xc, ws[...],
                       preferred_element_type=jnp.float32)
        iden = (iden + bs[...]).astype(jnp.bfloat16)
        h3 = h3 + iden.astype(jnp.float32)
    else:
        h3 = h3 + xc.astype(jnp.float32)
    return jnp.maximum(h3, 0.0).astype(jnp.bfloat16)


def _stage_body(*refs, cfgs, G, H0, has_xs):
    x_ref = refs[0]
    i = 1
    xs_ref = None
    if has_xs:
        xs_ref = refs[i]
        i += 1
    wrs = []
    for cfg in cfgs:
        n = 8 if cfg[4] else 6
        wrs.append(refs[i:i + n])
        i += n
    o_ref = refs[i]
    scr = refs[i + 1:]
    names = []
    if any(c[5] != 1 for c in cfgs):
        names += ['h1p32', 's32']
    names += ['h1pb', 'sb', 'h2']
    sc = dict(zip(names, scr, strict=True))

    H = H0
    xc = x_ref[...].reshape(G * H * H, x_ref.shape[3])
    for (Cin, width, nt, outc, has_ds, stride) in cfgs:
        xs = None
        if stride != 1 and has_ds:
            Ho = H // stride
            xs = xs_ref[...].reshape(G * Ho * Ho, Cin)
        xc = _bneck(xc, xs, wrs.pop(0), sc, G, H, Cin, width, nt, outc,
                    stride)
        H = H // stride
    o_ref[...] = xc.reshape(G, H, H, cfgs[-1][3])


@functools.lru_cache(maxsize=None)
def _build_stage(B, G, H0, cfgs):
    cfgs = tuple(cfgs)
    has_xs = any(c[4] and c[5] != 1 for c in cfgs)
    body = functools.partial(_stage_body, cfgs=cfgs, G=G, H0=H0,
                             has_xs=has_xs)
    in_specs = [pl.BlockSpec((G, H0, H0, cfgs[0][0]),
                             lambda i: (i, 0, 0, 0))]
    if has_xs:
        Ho0 = H0 // cfgs[0][5]
        in_specs.append(pl.BlockSpec((G, Ho0, Ho0, cfgs[0][0]),
                                     lambda i: (i, 0, 0, 0)))
    for (Cin, width, nt, outc, has_ds, stride) in cfgs:
        in_specs += [
            pl.BlockSpec((Cin, width), lambda i: (0, 0)),
            pl.BlockSpec((1, width), lambda i: (0, 0)),
            pl.BlockSpec((nt, 9 * _CB, _CB), lambda i: (0, 0, 0)),
            pl.BlockSpec((1, width), lambda i: (0, 0)),
            pl.BlockSpec((width, outc), lambda i: (0, 0)),
            pl.BlockSpec((1, outc), lambda i: (0, 0)),
        ]
        if has_ds:
            in_specs += [pl.BlockSpec((Cin, outc), lambda i: (0, 0)),
                         pl.BlockSpec((1, outc), lambda i: (0, 0))]
    Hlast = H0
    for c in cfgs:
        Hlast //= c[5]
    outc_last = cfgs[-1][3]
    scratch = []
    if any(c[5] != 1 for c in cfgs):
        Ho0 = H0 // cfgs[0][5]
        scratch += [
            pltpu.VMEM((G, H0 + 2, H0 + 2, _CB), jnp.float32),
            pltpu.VMEM((3, G, H0 + 2, Ho0, _CB), jnp.float32),
        ]
    H1 = Hlast
    scratch += [
        pltpu.VMEM((G, H1 + 2, H1 + 2, _CB), jnp.bfloat16),
        pltpu.VMEM((3, G, H1 + 2, H1, _CB), jnp.bfloat16),
        pltpu.VMEM((G * Hlast * Hlast, max(c[1] for c in cfgs)),
                   jnp.bfloat16),
    ]
    return pl.pallas_call(
        body,
        out_shape=jax.ShapeDtypeStruct((B, Hlast, Hlast, outc_last),
                                       jnp.bfloat16),
        grid=(B // G,),
        in_specs=in_specs,
        out_specs=pl.BlockSpec((G, Hlast, Hlast, outc_last),
                               lambda i: (i, 0, 0, 0)),
        scratch_shapes=scratch,
        compiler_params=pltpu.CompilerParams(
            dimension_semantics=("parallel",),
            vmem_limit_bytes=_VMEM_LIMIT),
    )


def _run_stage(x, blocks, G):
    B, H0, _, Cin0 = x.shape
    cfgs = []
    args = [x]
    cin = Cin0
    first_stride = blocks[0][-1]
    if first_stride != 1:
        args.append(x[:, ::first_stride, ::first_stride, :])
    wargs = []
    for (w1, b1, w2, b2, w3, b3, ws, bs, width, nt, outc, stride) in blocks:
        cfgs.append((cin, width, nt, outc, ws is not None, stride))
        wargs += [w1, b1.reshape(1, width), w2.reshape(nt, 9 * _CB, _CB),
                  b2.reshape(1, width), w3, b3.reshape(1, outc)]
        if ws is not None:
            wargs += [ws, bs.reshape(1, outc)]
        cin = outc
    call = _build_stage(B, G, H0, tuple(cfgs))
    return call(*(args + wargs))


def _stem_body(p_ref, w_ref, b_ref, o_ref, hp_ref):
    p = p_ref[0]
    h = jax.lax.dot_general(p, w_ref[...], (((0,), (0,)), ((), ())),
                            preferred_element_type=jnp.float32)
    h = jnp.maximum(h + b_ref[...], 0.0)
    hp_ref[0:1, :, :] = jnp.zeros((1, 114, 64), jnp.float32)
    hp_ref[113:114, :, :] = jnp.zeros((1, 114, 64), jnp.float32)
    hp_ref[1:113, 0:1, :] = jnp.zeros((112, 1, 64), jnp.float32)
    hp_ref[1:113, 113:114, :] = jnp.zeros((112, 1, 64), jnp.float32)
    hp_ref[1:113, 1:113, :] = h.reshape(112, 112, 64)
    m = hp_ref[0:111:2, 0:111:2, :]
    for kh in range(3):
        for kw in range(3):
            if kh == 0 and kw == 0:
                continue
            m = jnp.maximum(m, hp_ref[kh:kh + 111:2, kw:kw + 111:2, :])
    o_ref[...] = m.astype(jnp.bfloat16)[None]


@functools.lru_cache(maxsize=None)
def _build_stem(B):
    return pl.pallas_call(
        _stem_body,
        out_shape=jax.ShapeDtypeStruct((B, 56, 56, 64), jnp.bfloat16),
        grid=(B,),
        in_specs=[
            pl.BlockSpec((1, 49, 112 * 112), lambda i: (i, 0, 0)),
            pl.BlockSpec((49, 64), lambda i: (0, 0)),
            pl.BlockSpec((1, 64), lambda i: (0, 0)),
        ],
        out_specs=pl.BlockSpec((1, 56, 56, 64), lambda i: (i, 0, 0, 0)),
        scratch_shapes=[pltpu.VMEM((114, 114, 64), jnp.float32)],
        compiler_params=pltpu.CompilerParams(
            dimension_semantics=("parallel",),
            vmem_limit_bytes=_VMEM_LIMIT),
    )


def _head_body(x_ref, w_ref, b_ref, dense_ref, cls_ref):
    xv = x_ref[...].astype(jnp.float32)
    d = jnp.mean(xv, axis=1)
    dense_ref[...] = d
    cls = jnp.dot(d.astype(jnp.bfloat16), w_ref[...],
                  preferred_element_type=jnp.float32)
    cls_ref[...] = cls + b_ref[...]


@functools.lru_cache(maxsize=None)
def _build_head(B):
    return pl.pallas_call(
        _head_body,
        out_shape=(jax.ShapeDtypeStruct((B, 2048), jnp.float32),
                   jax.ShapeDtypeStruct((B, 6), jnp.float32)),
        grid=(1,),
        in_specs=[
            pl.BlockSpec((B, 49, 2048), lambda i: (0, 0, 0)),
            pl.BlockSpec((2048, 6), lambda i: (0, 0)),
            pl.BlockSpec((1, 6), lambda i: (0, 0)),
        ],
        out_specs=(pl.BlockSpec((B, 2048), lambda i: (0, 0)),
                   pl.BlockSpec((B, 6), lambda i: (0, 0))),
        compiler_params=pltpu.CompilerParams(
            dimension_semantics=("arbitrary",),
            vmem_limit_bytes=_VMEM_LIMIT),
    )


def kernel(stem_w2d, stem_b, s0b0_conv1_w2d, s0b0_conv1_b, s0b0_conv2_w2d, s0b0_conv2_b, s0b0_conv3_w2d, s0b0_conv3_b, s0b0_ds_w2d, s0b0_ds_b, s0b1_conv1_w2d, s0b1_conv1_b, s0b1_conv2_w2d, s0b1_conv2_b, s0b1_conv3_w2d, s0b1_conv3_b, s0b2_conv1_w2d, s0b2_conv1_b, s0b2_conv2_w2d, s0b2_conv2_b, s0b2_conv3_w2d, s0b2_conv3_b, s1b0_conv1_w2d, s1b0_conv1_b, s1b0_conv2_w2d, s1b0_conv2_b, s1b0_conv3_w2d, s1b0_conv3_b, s1b0_ds_w2d, s1b0_ds_b, s1b1_conv1_w2d, s1b1_conv1_b, s1b1_conv2_w2d, s1b1_conv2_b, s1b1_conv3_w2d, s1b1_conv3_b, s1b2_conv1_w2d, s1b2_conv1_b, s1b2_conv2_w2d, s1b2_conv2_b, s1b2_conv3_w2d, s1b2_conv3_b, s1b3_conv1_w2d, s1b3_conv1_b, s1b3_conv2_w2d, s1b3_conv2_b, s1b3_conv3_w2d, s1b3_conv3_b, s2b0_conv1_w2d, s2b0_conv1_b, s2b0_conv2_w2d, s2b0_conv2_b, s2b0_conv3_w2d, s2b0_conv3_b, s2b0_ds_w2d, s2b0_ds_b, s2b1_conv1_w2d, s2b1_conv1_b, s2b1_conv2_w2d, s2b1_conv2_b, s2b1_conv3_w2d, s2b1_conv3_b, s2b2_conv1_w2d, s2b2_conv1_b, s2b2_conv2_w2d, s2b2_conv2_b, s2b2_conv3_w2d, s2b2_conv3_b, s2b3_conv1_w2d, s2b3_conv1_b, s2b3_conv2_w2d, s2b3_conv2_b, s2b3_conv3_w2d, s2b3_conv3_b, s2b4_conv1_w2d, s2b4_conv1_b, s2b4_conv2_w2d, s2b4_conv2_b, s2b4_conv3_w2d, s2b4_conv3_b, s2b5_conv1_w2d, s2b5_conv1_b, s2b5_conv2_w2d, s2b5_conv2_b, s2b5_conv3_w2d, s2b5_conv3_b, s3b0_conv1_w2d, s3b0_conv1_b, s3b0_conv2_w2d, s3b0_conv2_b, s3b0_conv3_w2d, s3b0_conv3_b, s3b0_ds_w2d, s3b0_ds_b, s3b1_conv1_w2d, s3b1_conv1_b, s3b1_conv2_w2d, s3b1_conv2_b, s3b1_conv3_w2d, s3b1_conv3_b, s3b2_conv1_w2d, s3b2_conv1_b, s3b2_conv2_w2d, s3b2_conv2_b, s3b2_conv3_w2d, s3b2_conv3_b, fc_w, fc_b, x_nchw):
    B = x_nchw.shape[0]

    xb = x_nchw.reshape(B, 224, 224).astype(jnp.bfloat16)
    xpad = jnp.pad(xb, ((0, 0), (3, 3), (3, 3)))
    par = [[xpad[:, p::2, q::2] for q in (0, 1)] for p in (0, 1)]
    cols = []
    for kh in range(7):
        for kw in range(7):
            cols.append(par[kh % 2][kw % 2][:, kh // 2:kh // 2 + 112,
                                            kw // 2:kw // 2 + 112])
    patches = jnp.stack(cols, axis=1).reshape(B, 49, 112 * 112)
    x = _build_stem(B)(patches, stem_w2d, stem_b.reshape(1, 64))

    stage0 = [
        (s0b0_conv1_w2d, s0b0_conv1_b, s0b0_conv2_w2d, s0b0_conv2_b,
         s0b0_conv3_w2d, s0b0_conv3_b, s0b0_ds_w2d, s0b0_ds_b,
         128, 1, 256, 1),
        (s0b1_conv1_w2d, s0b1_conv1_b, s0b1_conv2_w2d, s0b1_conv2_b,
         s0b1_conv3_w2d, s0b1_conv3_b, None, None, 128, 1, 256, 1),
        (s0b2_conv1_w2d, s0b2_conv1_b, s0b2_conv2_w2d, s0b2_conv2_b,
         s0b2_conv3_w2d, s0b2_conv3_b, None, None, 128, 1, 256, 1),
    ]
    stage1 = [
        (s1b0_conv1_w2d, s1b0_conv1_b, s1b0_conv2_w2d, s1b0_conv2_b,
         s1b0_conv3_w2d, s1b0_conv3_b, s1b0_ds_w2d, s1b0_ds_b,
         256, 2, 512, 2),
        (s1b1_conv1_w2d, s1b1_conv1_b, s1b1_conv2_w2d, s1b1_conv2_b,
         s1b1_conv3_w2d, s1b1_conv3_b, None, None, 256, 2, 512, 1),
        (s1b2_conv1_w2d, s1b2_conv1_b, s1b2_conv2_w2d, s1b2_conv2_b,
         s1b2_conv3_w2d, s1b2_conv3_b, None, None, 256, 2, 512, 1),
        (s1b3_conv1_w2d, s1b3_conv1_b, s1b3_conv2_w2d, s1b3_conv2_b,
         s1b3_conv3_w2d, s1b3_conv3_b, None, None, 256, 2, 512, 1),
    ]
    stage2a = [
        (s2b0_conv1_w2d, s2b0_conv1_b, s2b0_conv2_w2d, s2b0_conv2_b,
         s2b0_conv3_w2d, s2b0_conv3_b, s2b0_ds_w2d, s2b0_ds_b,
         512, 4, 1024, 2),
        (s2b1_conv1_w2d, s2b1_conv1_b, s2b1_conv2_w2d, s2b1_conv2_b,
         s2b1_conv3_w2d, s2b1_conv3_b, None, None, 512, 4, 1024, 1),
        (s2b2_conv1_w2d, s2b2_conv1_b, s2b2_conv2_w2d, s2b2_conv2_b,
         s2b2_conv3_w2d, s2b2_conv3_b, None, None, 512, 4, 1024, 1),
    ]
    stage2b = [
        (s2b3_conv1_w2d, s2b3_conv1_b, s2b3_conv2_w2d, s2b3_conv2_b,
         s2b3_conv3_w2d, s2b3_conv3_b, None, None, 512, 4, 1024, 1),
        (s2b4_conv1_w2d, s2b4_conv1_b, s2b4_conv2_w2d, s2b4_conv2_b,
         s2b4_conv3_w2d, s2b4_conv3_b, None, None, 512, 4, 1024, 1),
        (s2b5_conv1_w2d, s2b5_conv1_b, s2b5_conv2_w2d, s2b5_conv2_b,
         s2b5_conv3_w2d, s2b5_conv3_b, None, None, 512, 4, 1024, 1),
    ]
    stage3a = [
        (s3b0_conv1_w2d, s3b0_conv1_b, s3b0_conv2_w2d, s3b0_conv2_b,
         s3b0_conv3_w2d, s3b0_conv3_b, s3b0_ds_w2d, s3b0_ds_b,
         1024, 8, 2048, 2),
    ]
    stage3b = [
        (s3b1_conv1_w2d, s3b1_conv1_b, s3b1_conv2_w2d, s3b1_conv2_b,
         s3b1_conv3_w2d, s3b1_conv3_b, None, None, 1024, 8, 2048, 1),
        (s3b2_conv1_w2d, s3b2_conv1_b, s3b2_conv2_w2d, s3b2_conv2_b,
         s3b2_conv3_w2d, s3b2_conv3_b, None, None, 1024, 8, 2048, 1),
    ]
    for i, blk in enumerate(stage0):
        x = _run_stage(x, [blk], G=1)
    for i, blk in enumerate(stage1):
        x = _run_stage(x, [blk], G=2)
    for i, blk in enumerate(stage2a + stage2b):
        x = _run_stage(x, [blk], G=4)
    x = _run_stage(x, stage3a, G=4)
    for i, blk in enumerate(stage3b):
        x = _run_stage(x, [blk], G=8)

    dense, cls = _build_head(B)(x.reshape(B, 49, 2048),
                                fc_w.astype(jnp.bfloat16),
                                fc_b.reshape(1, 6))
    return dense, cls
